# baseline (speedup 1.0000x reference)
import jax
import jax.numpy as jnp
from jax import lax
from jax.experimental import pallas as pl
from jax.experimental.pallas import tpu as pltpu
from jax.experimental.pallas import tpu_sc as plsc

N = 10000
E = 320000
P = 100000
D = 128
HID = 128
OUT = 128

NS = 16
L = 16

EC = E // NS
CI = 2000
C = 80
NBUF = 3
ACC_CHUNKS = ([(k * C, C) for k in range(N // C)] +
              ([((N // C) * C, N % C)] if N % C else []))
SBLK = 50
SBI = SBLK * C
PP = 102400
PEC = PP // NS
PC = 128

BLK = 2000


def _mesh():
    return plsc.VectorSubcoreMesh(core_axis_name="c", subcore_axis_name="s")


def _fill_const(ref, n, val, dtype):
    def body(i, _):
        ref[pl.ds(i * L, L)] = jnp.full((L,), val, dtype)
        return 0
    lax.fori_loop(0, n // L, body, 0)


def _deg_body(src_f, dst_f, src_s, dst_s, deg0, deg1, deg2, deg3,
              acc_a, acc_b, idx0, idx1, ones_v, zero_v, sem0, sem1):
    c = lax.axis_index("c")
    s = lax.axis_index("s")
    _fill_const(ones_v, CI, 1.0, jnp.float32)

    @pl.when(s == 0)
    def _():
        _fill_const(zero_v, N, 0.0, jnp.float32)
        pltpu.sync_copy(zero_v, acc_a)
        pltpu.sync_copy(zero_v, acc_b)

    plsc.subcore_barrier()

    def run(src_ref, dst_ref):
        nch = EC // CI
        chunks = [(e_ref, acc, j)
                  for j in range(nch)
                  for (e_ref, acc) in ((src_ref, acc_a), (dst_ref, acc_b))]
        bufs = ((idx0, sem0), (idx1, sem1))
        for k, (e_ref, acc, j) in enumerate(chunks):
            idx, sem = bufs[k % 2]
            if k >= 2:
                pe, pacc, pj = chunks[k - 2]
                pltpu.make_async_copy(ones_v, pacc.at[idx], sem).wait()
            pltpu.sync_copy(e_ref.at[pl.ds(s * EC + j * CI, CI)], idx)
            pltpu.async_copy(ones_v, acc.at[idx], sem, add=True)
        for k in (len(chunks) - 2, len(chunks) - 1):
            e_ref, acc, j = chunks[k]
            idx, sem = bufs[k % 2]
            pltpu.make_async_copy(ones_v, acc.at[idx], sem).wait()

    @pl.when(c == 0)
    def _():
        run(src_f, dst_f)

    @pl.when(c == 1)
    def _():
        run(src_s, dst_s)

    plsc.subcore_barrier()

    @pl.when(s == 0)
    def _():
        @pl.when(c == 0)
        def _():
            pltpu.sync_copy(acc_a, deg0)
            pltpu.sync_copy(acc_b, deg1)

        @pl.when(c == 1)
        def _():
            pltpu.sync_copy(acc_a, deg2)
            pltpu.sync_copy(acc_b, deg3)


def _sc_deg(src_f, dst_f, src_s, dst_s):
    return pl.kernel(
        _deg_body,
        out_type=[jax.ShapeDtypeStruct((N,), jnp.float32)] * 4,
        mesh=_mesh(),
        scratch_types=[
            pltpu.VMEM_SHARED((N,), jnp.float32),
            pltpu.VMEM_SHARED((N,), jnp.float32),
            pltpu.VMEM((CI,), jnp.int32),
            pltpu.VMEM((CI,), jnp.int32),
            pltpu.VMEM((CI,), jnp.float32),
            pltpu.VMEM((N,), jnp.float32),
            pltpu.SemaphoreType.DMA,
            pltpu.SemaphoreType.DMA,
        ],
    )(src_f, dst_f, src_s, dst_s)


def _segsum_body(x_f, x_s, src_f, dst_f, src_s, dst_s, out_f, out_s,
                 acc, rows0, rows1, rows2, sblk0, sblk1, didx0, didx1, didx2,
                 sem_g0, sem_g1, sem_g2, sem_s0, sem_s1, sem_s2):
    c = lax.axis_index("c")
    s = lax.axis_index("s")
    bufs = ((rows0, didx0, sem_g0, sem_s0),
            (rows1, didx1, sem_g1, sem_s1),
            (rows2, didx2, sem_g2, sem_s2))
    sblks = (sblk0, sblk1)
    nch = EC // C

    def zero_rows(r, _):
        for k in range(D // L):
            rows0[r, pl.ds(k * L, L)] = jnp.zeros((L,), jnp.float32)
        return 0
    lax.fori_loop(0, C, zero_rows, 0)

    for k, (off, sz) in enumerate(ACC_CHUNKS):
        @pl.when(s == k % NS)
        def _():
            pltpu.async_copy(rows0.at[pl.ds(0, sz)], acc.at[pl.ds(off, sz)],
                             sem_s0)
    for k, (off, sz) in enumerate(ACC_CHUNKS):
        @pl.when(s == k % NS)
        def _():
            pltpu.make_async_copy(rows0.at[pl.ds(0, sz)],
                                  acc.at[pl.ds(off, sz)], sem_s0).wait()

    plsc.subcore_barrier()

    def run(x_ref, src_ref, dst_ref):
        def start_gather(j2, b):
            rows, didx, sem_g, _ = bufs[b]
            @pl.when(j2 % SBLK == 0)
            def _():
                for q in range(2):
                    @pl.when((j2 // SBLK) % 2 == q)
                    def _():
                        pltpu.sync_copy(
                            src_ref.at[pl.ds(s * EC + (j2 // SBLK) * SBI,
                                             SBI)], sblks[q])
            off = (j2 % SBLK) * C
            for q in range(2):
                @pl.when((j2 // SBLK) % 2 == q)
                def _():
                    pltpu.async_copy(x_ref.at[sblks[q].at[pl.ds(off, C)]],
                                     rows, sem_g)

        def finish(j, b):
            rows, didx, sem_g, sem_s = bufs[b]
            base = s * EC + j * C
            pltpu.sync_copy(dst_ref.at[pl.ds(base, C)], didx)
            pltpu.make_async_copy(x_ref.at[didx], rows, sem_g).wait()
            pltpu.async_copy(rows, acc.at[didx], sem_s, add=True)
            pltpu.make_async_copy(rows, acc.at[didx], sem_s).wait()

        for b in range(NBUF):
            start_gather(b, b)

        def body(j, _):
            for b in range(NBUF):
                @pl.when(j % NBUF == b)
                def _():
                    finish(j, b)
                    start_gather(j + NBUF, b)
            return 0
        lax.fori_loop(0, nch - NBUF, body, 0)
        for j in range(nch - NBUF, nch):
            finish(j, j % NBUF)

    @pl.when(c == 0)
    def _():
        run(x_f, src_f, dst_f)

    @pl.when(c == 1)
    def _():
        run(x_s, src_s, dst_s)

    plsc.subcore_barrier()

    def writeout(out_ref):
        for k, (off, sz) in enumerate(ACC_CHUNKS):
            @pl.when(s == k % NS)
            def _():
                pltpu.async_copy(acc.at[pl.ds(off, sz)],
                                 out_ref.at[pl.ds(off, sz)], sem_g0)
        for k, (off, sz) in enumerate(ACC_CHUNKS):
            @pl.when(s == k % NS)
            def _():
                pltpu.make_async_copy(acc.at[pl.ds(off, sz)],
                                      out_ref.at[pl.ds(off, sz)],
                                      sem_g0).wait()

    @pl.when(c == 0)
    def _():
        writeout(out_f)

    @pl.when(c == 1)
    def _():
        writeout(out_s)


def _sc_segsum(x_f, x_s, src_f, dst_f, src_s, dst_s):
    return pl.kernel(
        _segsum_body,
        out_type=[jax.ShapeDtypeStruct((N, D), jnp.float32),
                  jax.ShapeDtypeStruct((N, D), jnp.float32)],
        mesh=_mesh(),
        scratch_types=[
            pltpu.VMEM_SHARED((N, D), jnp.float32),
            pltpu.VMEM((C, D), jnp.float32),
            pltpu.VMEM((C, D), jnp.float32),
            pltpu.VMEM((C, D), jnp.float32),
            pltpu.VMEM((SBI,), jnp.int32),
            pltpu.VMEM((SBI,), jnp.int32),
            pltpu.VMEM((C,), jnp.int32),
            pltpu.VMEM((C,), jnp.int32),
            pltpu.VMEM((C,), jnp.int32),
            pltpu.SemaphoreType.DMA,
            pltpu.SemaphoreType.DMA,
            pltpu.SemaphoreType.DMA,
            pltpu.SemaphoreType.DMA,
            pltpu.SemaphoreType.DMA,
            pltpu.SemaphoreType.DMA,
        ],
    )(x_f, x_s, src_f, dst_f, src_s, dst_s)


def _predict_body(hn_f, hn_s, p0, p1, n0, n1, pa, pc, na, nc,
                  table, rows0, rows1, ei0, ei1,
                  sem_g0, sem_g1, sem_w):
    c = lax.axis_index("c")
    s = lax.axis_index("s")
    bufs = ((rows0, sem_g0), (rows1, sem_g1))
    nch = PEC // PC

    def load_table(h):
        for k, (off, sz) in enumerate(ACC_CHUNKS):
            @pl.when(s == k % NS)
            def _():
                pltpu.async_copy(h.at[pl.ds(off, sz)],
                                 table.at[pl.ds(off, sz)], sem_w)
        for k, (off, sz) in enumerate(ACC_CHUNKS):
            @pl.when(s == k % NS)
            def _():
                pltpu.make_async_copy(h.at[pl.ds(off, sz)],
                                      table.at[pl.ds(off, sz)], sem_w).wait()

    def run(ei, o):
        def start_gather(j, b):
            rows, sem_g = bufs[b]
            pltpu.async_copy(table.at[ei.at[pl.ds(j * PC, PC)]], rows, sem_g)

        def finish(j, b):
            rows, sem_g = bufs[b]
            base = s * PEC + j * PC
            pltpu.make_async_copy(table.at[ei.at[pl.ds(j * PC, PC)]],
                                  rows, sem_g).wait()
            pltpu.async_copy(rows, o.at[pl.ds(base, PC)], sem_w)
            pltpu.make_async_copy(rows, o.at[pl.ds(base, PC)], sem_w).wait()

        start_gather(0, 0)
        start_gather(1, 1)

        def body(j, _):
            for b in range(2):
                @pl.when(j % 2 == b)
                def _():
                    finish(j, b)
                    start_gather(j + 2, b)
            return 0
        lax.fori_loop(0, nch - 2, body, 0)
        for j in (nch - 2, nch - 1):
            finish(j, j % 2)

    def core(h, e_pos, e_neg, o_pos, o_neg):
        load_table(h)
        pltpu.sync_copy(e_pos.at[pl.ds(s * PEC, PEC)], ei0)
        pltpu.sync_copy(e_neg.at[pl.ds(s * PEC, PEC)], ei1)
        plsc.subcore_barrier()
        run(ei0, o_pos)
        run(ei1, o_neg)

    @pl.when(c == 0)
    def _():
        core(hn_f, p0, n0, pa, na)

    @pl.when(c == 1)
    def _():
        core(hn_s, p1, n1, pc, nc)


def _sc_predict_gather(hn_f, hn_s, p0, p1, n0, n1):
    return pl.kernel(
        _predict_body,
        out_type=[jax.ShapeDtypeStruct((PP, D), jnp.float32)] * 4,
        mesh=_mesh(),
        scratch_types=[
            pltpu.VMEM_SHARED((N, D), jnp.float32),
            pltpu.VMEM((PC, D), jnp.float32),
            pltpu.VMEM((PC, D), jnp.float32),
            pltpu.VMEM((PEC,), jnp.int32),
            pltpu.VMEM((PEC,), jnp.int32),
            pltpu.SemaphoreType.DMA,
            pltpu.SemaphoreType.DMA,
            pltpu.SemaphoreType.DMA,
        ],
    )(hn_f, hn_s, p0, p1, n0, n1)


def _row_spec():
    return pl.BlockSpec((BLK, D), lambda i: (i, 0))


def _col_spec():
    return pl.BlockSpec((BLK, 1), lambda i: (i, 0))


def _w_spec():
    return pl.BlockSpec((D, D), lambda i: (0, 0))


def _b_spec():
    return pl.BlockSpec((D,), lambda i: (0,))


def _scale_body(xf_ref, df_ref, xs_ref, ds_ref, of_ref, os_ref):
    of_ref[...] = xf_ref[...] * lax.rsqrt(jnp.maximum(df_ref[...], 1.0))
    os_ref[...] = xs_ref[...] * lax.rsqrt(jnp.maximum(ds_ref[...], 1.0))


def _tc_scale(xf, df, xs, ds):
    return pl.pallas_call(
        _scale_body,
        grid=(N // BLK,),
        in_specs=[_row_spec(), _col_spec(), _row_spec(), _col_spec()],
        out_specs=[_row_spec(), _row_spec()],
        out_shape=[jax.ShapeDtypeStruct((N, D), jnp.float32)] * 2,
    )(xf, df, xs, ds)


def _layer_graph(m, rin, rout, w, b):
    h = jnp.dot(m * rin, w, preferred_element_type=jnp.float32)
    h = jnp.maximum(h + b, 0.0)
    return h, h * rout


def _layer_body(mf_ref, dinf_ref, doutf_ref, ms_ref, dins_ref, douts_ref,
                w_ref, b_ref, hf_ref, xf_ref, hs_ref, xs_ref):
    w = w_ref[...]
    b = b_ref[...]
    hf_ref[...], xf_ref[...] = _layer_graph(
        mf_ref[...], lax.rsqrt(jnp.maximum(dinf_ref[...], 1.0)),
        lax.rsqrt(jnp.maximum(doutf_ref[...], 1.0)), w, b)
    hs_ref[...], xs_ref[...] = _layer_graph(
        ms_ref[...], lax.rsqrt(jnp.maximum(dins_ref[...], 1.0)),
        lax.rsqrt(jnp.maximum(douts_ref[...], 1.0)), w, b)


def _tc_layer(mf, dinf, doutf, ms, dins, douts, w, b):
    return pl.pallas_call(
        _layer_body,
        grid=(N // BLK,),
        in_specs=[_row_spec(), _col_spec(), _col_spec(),
                  _row_spec(), _col_spec(), _col_spec(),
                  _w_spec(), _b_spec()],
        out_specs=[_row_spec()] * 4,
        out_shape=[jax.ShapeDtypeStruct((N, D), jnp.float32)] * 4,
    )(mf, dinf, doutf, ms, dins, douts, w, b)


def _final_graph(m2, rin, x, h1, w1, b1, wp, bp):
    h2 = jnp.dot(m2 * rin, w1, preferred_element_type=jnp.float32)
    h2 = jnp.maximum(h2 + b1, 0.0)
    h = (jnp.dot(x, wp[0], preferred_element_type=jnp.float32)
         + jnp.dot(h1, wp[1], preferred_element_type=jnp.float32)
         + jnp.dot(h2, wp[2], preferred_element_type=jnp.float32)
         + bp)
    nrm = lax.rsqrt(jnp.sum(h * h, axis=1, keepdims=True))
    return h, h * nrm


def _final_body(m2f_ref, dinf_ref, xf_ref, h1f_ref,
                m2s_ref, dins_ref, xs_ref, h1s_ref,
                w1_ref, b1_ref, wp1a_ref, wp1b_ref, wp1c_ref, bp1_ref,
                wp2a_ref, wp2b_ref, wp2c_ref, bp2_ref,
                hf_ref, hnf_ref, hs_ref, hns_ref):
    w1 = w1_ref[...]
    b1 = b1_ref[...]
    hf_ref[...], hnf_ref[...] = _final_graph(
        m2f_ref[...], lax.rsqrt(jnp.maximum(dinf_ref[...], 1.0)),
        xf_ref[...], h1f_ref[...], w1, b1,
        (wp1a_ref[...], wp1b_ref[...], wp1c_ref[...]), bp1_ref[...])
    hs_ref[...], hns_ref[...] = _final_graph(
        m2s_ref[...], lax.rsqrt(jnp.maximum(dins_ref[...], 1.0)),
        xs_ref[...], h1s_ref[...], w1, b1,
        (wp2a_ref[...], wp2b_ref[...], wp2c_ref[...]), bp2_ref[...])


def _tc_final(m2f, dinf, xf, h1f, m2s, dins, xs, h1s,
              w1, b1, wp1, bp1, wp2, bp2):
    return pl.pallas_call(
        _final_body,
        grid=(N // BLK,),
        in_specs=[_row_spec(), _col_spec(), _row_spec(), _row_spec(),
                  _row_spec(), _col_spec(), _row_spec(), _row_spec(),
                  _w_spec(), _b_spec(),
                  _w_spec(), _w_spec(), _w_spec(), _b_spec(),
                  _w_spec(), _w_spec(), _w_spec(), _b_spec()],
        out_specs=[_row_spec()] * 4,
        out_shape=[jax.ShapeDtypeStruct((N, D), jnp.float32)] * 4,
    )(m2f, dinf, xf, h1f, m2s, dins, xs, h1s,
      w1, b1, wp1[0], wp1[1], wp1[2], bp1, wp2[0], wp2[1], wp2[2], bp2)


PBLK = 4096


def _dot_body(pa_ref, pc_ref, na_ref, nc_ref, po_ref, no_ref):
    po_ref[...] = jnp.sum(pa_ref[...] * pc_ref[...], axis=1, keepdims=True)
    no_ref[...] = jnp.sum(na_ref[...] * nc_ref[...], axis=1, keepdims=True)


def _tc_dot(pa, pc, na, nc):
    rs = pl.BlockSpec((PBLK, D), lambda i: (i, 0))
    cs = pl.BlockSpec((PBLK, 1), lambda i: (i, 0))
    return pl.pallas_call(
        _dot_body,
        grid=(PP // PBLK,),
        in_specs=[rs, rs, rs, rs],
        out_specs=[cs, cs],
        out_shape=[jax.ShapeDtypeStruct((PP, 1), jnp.float32)] * 2,
    )(pa, pc, na, nc)


def kernel(x_first, x_second, edge_index_first, edge_index_second,
           edges_positive_supervision, edges_negative_supervision,
           W0, b0, W1, b1, Wp1, bp1, Wp2, bp2):
    src_f = edge_index_first[0]
    dst_f = edge_index_first[1]
    src_s = edge_index_second[0]
    dst_s = edge_index_second[1]

    deg0, deg1, deg2, deg3 = _sc_deg(src_f, dst_f, src_s, dst_s)
    d_out_f = deg0.reshape(N, 1)
    d_in_f = deg1.reshape(N, 1)
    d_out_s = deg2.reshape(N, 1)
    d_in_s = deg3.reshape(N, 1)

    x0f, x0s = _tc_scale(x_first, d_out_f, x_second, d_out_s)
    m0f, m0s = _sc_segsum(x0f, x0s, src_f, dst_f, src_s, dst_s)
    h1f, x1f, h1s, x1s = _tc_layer(m0f, d_in_f, d_out_f,
                                   m0s, d_in_s, d_out_s, W0, b0)
    m1f, m1s = _sc_segsum(x1f, x1s, src_f, dst_f, src_s, dst_s)
    wp1 = (Wp1[0:D], Wp1[D:2 * D], Wp1[2 * D:3 * D])
    wp2 = (Wp2[0:D], Wp2[D:2 * D], Wp2[2 * D:3 * D])
    hf, hnf, hs, hns = _tc_final(m1f, d_in_f, x_first, h1f,
                                 m1s, d_in_s, x_second, h1s,
                                 W1, b1, wp1, bp1, wp2, bp2)

    pad = ((0, PP - P),)
    p0 = jnp.pad(edges_positive_supervision[0], pad)
    p1 = jnp.pad(edges_positive_supervision[1], pad)
    n0 = jnp.pad(edges_negative_supervision[0], pad)
    n1 = jnp.pad(edges_negative_supervision[1], pad)
    pa, pc, na, nc = _sc_predict_gather(hnf, hns, p0, p1, n0, n1)
    pos, neg = _tc_dot(pa, pc, na, nc)
    return (pos[:P], neg[:P], hf, hs)


# speedup vs baseline: 1.1416x; 1.1416x over previous
import jax
import jax.numpy as jnp
from jax import lax
from jax.experimental import pallas as pl
from jax.experimental.pallas import tpu as pltpu
from jax.experimental.pallas import tpu_sc as plsc

N = 10000
E = 320000
P = 100000
D = 128
HID = 128
OUT = 128

NS = 16
L = 16

EC = E // NS
CI = 2000
C = 160
ACC_CHUNKS = [(k * C, C) for k in range(N // C)] + [((N // C) * C, N % C)]
SBLK = 25
SBI = SBLK * C
PP = 102400
PEC = PP // NS
PC = 128

BLK = 2000


def _mesh():
    return plsc.VectorSubcoreMesh(core_axis_name="c", subcore_axis_name="s")


def _fill_const(ref, n, val, dtype):
    def body(i, _):
        ref[pl.ds(i * L, L)] = jnp.full((L,), val, dtype)
        return 0
    lax.fori_loop(0, n // L, body, 0)


def _deg_body(src_f, dst_f, src_s, dst_s, deg0, deg1, deg2, deg3,
              acc_a, acc_b, idx0, idx1, ones_v, zero_v, sem0, sem1):
    c = lax.axis_index("c")
    s = lax.axis_index("s")
    _fill_const(ones_v, CI, 1.0, jnp.float32)

    @pl.when(s == 0)
    def _():
        _fill_const(zero_v, N, 0.0, jnp.float32)
        pltpu.sync_copy(zero_v, acc_a)
        pltpu.sync_copy(zero_v, acc_b)

    plsc.subcore_barrier()

    def run(src_ref, dst_ref):
        nch = EC // CI
        chunks = [(e_ref, acc, j)
                  for j in range(nch)
                  for (e_ref, acc) in ((src_ref, acc_a), (dst_ref, acc_b))]
        bufs = ((idx0, sem0), (idx1, sem1))
        for k, (e_ref, acc, j) in enumerate(chunks):
            idx, sem = bufs[k % 2]
            if k >= 2:
                pe, pacc, pj = chunks[k - 2]
                pltpu.make_async_copy(ones_v, pacc.at[idx], sem).wait()
            pltpu.sync_copy(e_ref.at[pl.ds(s * EC + j * CI, CI)], idx)
            pltpu.async_copy(ones_v, acc.at[idx], sem, add=True)
        for k in (len(chunks) - 2, len(chunks) - 1):
            e_ref, acc, j = chunks[k]
            idx, sem = bufs[k % 2]
            pltpu.make_async_copy(ones_v, acc.at[idx], sem).wait()

    @pl.when(c == 0)
    def _():
        run(src_f, dst_f)

    @pl.when(c == 1)
    def _():
        run(src_s, dst_s)

    plsc.subcore_barrier()

    @pl.when(s == 0)
    def _():
        @pl.when(c == 0)
        def _():
            pltpu.sync_copy(acc_a, deg0)
            pltpu.sync_copy(acc_b, deg1)

        @pl.when(c == 1)
        def _():
            pltpu.sync_copy(acc_a, deg2)
            pltpu.sync_copy(acc_b, deg3)


def _sc_deg(src_f, dst_f, src_s, dst_s):
    return pl.kernel(
        _deg_body,
        out_type=[jax.ShapeDtypeStruct((N,), jnp.float32)] * 4,
        mesh=_mesh(),
        scratch_types=[
            pltpu.VMEM_SHARED((N,), jnp.float32),
            pltpu.VMEM_SHARED((N,), jnp.float32),
            pltpu.VMEM((CI,), jnp.int32),
            pltpu.VMEM((CI,), jnp.int32),
            pltpu.VMEM((CI,), jnp.float32),
            pltpu.VMEM((N,), jnp.float32),
            pltpu.SemaphoreType.DMA,
            pltpu.SemaphoreType.DMA,
        ],
    )(src_f, dst_f, src_s, dst_s)


def _segsum_body(x_f, x_s, src_f, dst_f, src_s, dst_s, out_f, out_s,
                 acc, rows0, rows1, sblk0, sblk1, didx0, didx1,
                 sem_g0, sem_g1, sem_s0, sem_s1):
    c = lax.axis_index("c")
    s = lax.axis_index("s")
    bufs = ((rows0, didx0, sem_g0, sem_s0),
            (rows1, didx1, sem_g1, sem_s1))
    sblks = (sblk0, sblk1)
    nch = EC // C

    def zero_rows(r, _):
        for k in range(D // L):
            rows0[r, pl.ds(k * L, L)] = jnp.zeros((L,), jnp.float32)
        return 0
    lax.fori_loop(0, C, zero_rows, 0)

    for k, (off, sz) in enumerate(ACC_CHUNKS):
        @pl.when(s == k % NS)
        def _():
            pltpu.async_copy(rows0.at[pl.ds(0, sz)], acc.at[pl.ds(off, sz)],
                             sem_s0)
    for k, (off, sz) in enumerate(ACC_CHUNKS):
        @pl.when(s == k % NS)
        def _():
            pltpu.make_async_copy(rows0.at[pl.ds(0, sz)],
                                  acc.at[pl.ds(off, sz)], sem_s0).wait()

    plsc.subcore_barrier()

    def run(x_ref, src_ref, dst_ref):
        def start_gather(j2, b):
            rows, didx, sem_g, _ = bufs[b]
            @pl.when(j2 % SBLK == 0)
            def _():
                for q in range(2):
                    @pl.when((j2 // SBLK) % 2 == q)
                    def _():
                        pltpu.sync_copy(
                            src_ref.at[pl.ds(s * EC + (j2 // SBLK) * SBI,
                                             SBI)], sblks[q])
            off = (j2 % SBLK) * C
            for q in range(2):
                @pl.when((j2 // SBLK) % 2 == q)
                def _():
                    pltpu.async_copy(x_ref.at[sblks[q].at[pl.ds(off, C)]],
                                     rows, sem_g)

        def finish(j, b):
            rows, didx, sem_g, sem_s = bufs[b]
            base = s * EC + j * C
            pltpu.sync_copy(dst_ref.at[pl.ds(base, C)], didx)
            pltpu.make_async_copy(x_ref.at[didx], rows, sem_g).wait()
            pltpu.async_copy(rows, acc.at[didx], sem_s, add=True)
            pltpu.make_async_copy(rows, acc.at[didx], sem_s).wait()

        start_gather(0, 0)
        start_gather(1, 1)

        def body(j, _):
            for b in range(2):
                @pl.when(j % 2 == b)
                def _():
                    finish(j, b)
                    start_gather(j + 2, b)
            return 0
        lax.fori_loop(0, nch - 2, body, 0)
        for j in (nch - 2, nch - 1):
            finish(j, j % 2)

    @pl.when(c == 0)
    def _():
        run(x_f, src_f, dst_f)

    @pl.when(c == 1)
    def _():
        run(x_s, src_s, dst_s)

    plsc.subcore_barrier()

    def writeout(out_ref):
        for k, (off, sz) in enumerate(ACC_CHUNKS):
            @pl.when(s == k % NS)
            def _():
                pltpu.async_copy(acc.at[pl.ds(off, sz)],
                                 out_ref.at[pl.ds(off, sz)], sem_g0)
        for k, (off, sz) in enumerate(ACC_CHUNKS):
            @pl.when(s == k % NS)
            def _():
                pltpu.make_async_copy(acc.at[pl.ds(off, sz)],
                                      out_ref.at[pl.ds(off, sz)],
                                      sem_g0).wait()

    @pl.when(c == 0)
    def _():
        writeout(out_f)

    @pl.when(c == 1)
    def _():
        writeout(out_s)


def _sc_segsum(x_f, x_s, src_f, dst_f, src_s, dst_s):
    return pl.kernel(
        _segsum_body,
        out_type=[jax.ShapeDtypeStruct((N, D), jnp.float32),
                  jax.ShapeDtypeStruct((N, D), jnp.float32)],
        mesh=_mesh(),
        scratch_types=[
            pltpu.VMEM_SHARED((N, D), jnp.float32),
            pltpu.VMEM((C, D), jnp.float32),
            pltpu.VMEM((C, D), jnp.float32),
            pltpu.VMEM((SBI,), jnp.int32),
            pltpu.VMEM((SBI,), jnp.int32),
            pltpu.VMEM((C,), jnp.int32),
            pltpu.VMEM((C,), jnp.int32),
            pltpu.SemaphoreType.DMA,
            pltpu.SemaphoreType.DMA,
            pltpu.SemaphoreType.DMA,
            pltpu.SemaphoreType.DMA,
        ],
    )(x_f, x_s, src_f, dst_f, src_s, dst_s)


def _predict_body(hn_f, hn_s, p0, p1, n0, n1, pa, pc, na, nc,
                  table, rows0, rows1, ei0, ei1,
                  sem_g0, sem_g1, sem_w):
    c = lax.axis_index("c")
    s = lax.axis_index("s")
    bufs = ((rows0, sem_g0), (rows1, sem_g1))
    nch = PEC // PC

    def load_table(h):
        for k, (off, sz) in enumerate(ACC_CHUNKS):
            @pl.when(s == k % NS)
            def _():
                pltpu.async_copy(h.at[pl.ds(off, sz)],
                                 table.at[pl.ds(off, sz)], sem_w)
        for k, (off, sz) in enumerate(ACC_CHUNKS):
            @pl.when(s == k % NS)
            def _():
                pltpu.make_async_copy(h.at[pl.ds(off, sz)],
                                      table.at[pl.ds(off, sz)], sem_w).wait()

    def run(ei, o):
        def start_gather(j, b):
            rows, sem_g = bufs[b]
            pltpu.async_copy(table.at[ei.at[pl.ds(j * PC, PC)]], rows, sem_g)

        def finish(j, b):
            rows, sem_g = bufs[b]
            base = s * PEC + j * PC
            pltpu.make_async_copy(table.at[ei.at[pl.ds(j * PC, PC)]],
                                  rows, sem_g).wait()
            pltpu.async_copy(rows, o.at[pl.ds(base, PC)], sem_w)
            pltpu.make_async_copy(rows, o.at[pl.ds(base, PC)], sem_w).wait()

        start_gather(0, 0)
        start_gather(1, 1)

        def body(j, _):
            for b in range(2):
                @pl.when(j % 2 == b)
                def _():
                    finish(j, b)
                    start_gather(j + 2, b)
            return 0
        lax.fori_loop(0, nch - 2, body, 0)
        for j in (nch - 2, nch - 1):
            finish(j, j % 2)

    def core(h, e_pos, e_neg, o_pos, o_neg):
        load_table(h)
        pltpu.sync_copy(e_pos.at[pl.ds(s * PEC, PEC)], ei0)
        pltpu.sync_copy(e_neg.at[pl.ds(s * PEC, PEC)], ei1)
        plsc.subcore_barrier()
        run(ei0, o_pos)
        run(ei1, o_neg)

    @pl.when(c == 0)
    def _():
        core(hn_f, p0, n0, pa, na)

    @pl.when(c == 1)
    def _():
        core(hn_s, p1, n1, pc, nc)


def _sc_predict_gather(hn_f, hn_s, p0, p1, n0, n1):
    return pl.kernel(
        _predict_body,
        out_type=[jax.ShapeDtypeStruct((PP, D), jnp.float32)] * 4,
        mesh=_mesh(),
        scratch_types=[
            pltpu.VMEM_SHARED((N, D), jnp.float32),
            pltpu.VMEM((PC, D), jnp.float32),
            pltpu.VMEM((PC, D), jnp.float32),
            pltpu.VMEM((PEC,), jnp.int32),
            pltpu.VMEM((PEC,), jnp.int32),
            pltpu.SemaphoreType.DMA,
            pltpu.SemaphoreType.DMA,
            pltpu.SemaphoreType.DMA,
        ],
    )(hn_f, hn_s, p0, p1, n0, n1)


def _row_spec():
    return pl.BlockSpec((BLK, D), lambda i: (i, 0))


def _col_spec():
    return pl.BlockSpec((BLK, 1), lambda i: (i, 0))


def _w_spec():
    return pl.BlockSpec((D, D), lambda i: (0, 0))


def _b_spec():
    return pl.BlockSpec((D,), lambda i: (0,))


def _scale_body(xf_ref, df_ref, xs_ref, ds_ref, of_ref, os_ref):
    of_ref[...] = xf_ref[...] * lax.rsqrt(jnp.maximum(df_ref[...], 1.0))
    os_ref[...] = xs_ref[...] * lax.rsqrt(jnp.maximum(ds_ref[...], 1.0))


def _tc_scale(xf, df, xs, ds):
    return pl.pallas_call(
        _scale_body,
        grid=(N // BLK,),
        in_specs=[_row_spec(), _col_spec(), _row_spec(), _col_spec()],
        out_specs=[_row_spec(), _row_spec()],
        out_shape=[jax.ShapeDtypeStruct((N, D), jnp.float32)] * 2,
    )(xf, df, xs, ds)


def _layer_graph(m, rin, rout, w, b):
    h = jnp.dot(m * rin, w, preferred_element_type=jnp.float32)
    h = jnp.maximum(h + b, 0.0)
    return h, h * rout


def _layer_body(mf_ref, dinf_ref, doutf_ref, ms_ref, dins_ref, douts_ref,
                w_ref, b_ref, hf_ref, xf_ref, hs_ref, xs_ref):
    w = w_ref[...]
    b = b_ref[...]
    hf_ref[...], xf_ref[...] = _layer_graph(
        mf_ref[...], lax.rsqrt(jnp.maximum(dinf_ref[...], 1.0)),
        lax.rsqrt(jnp.maximum(doutf_ref[...], 1.0)), w, b)
    hs_ref[...], xs_ref[...] = _layer_graph(
        ms_ref[...], lax.rsqrt(jnp.maximum(dins_ref[...], 1.0)),
        lax.rsqrt(jnp.maximum(douts_ref[...], 1.0)), w, b)


def _tc_layer(mf, dinf, doutf, ms, dins, douts, w, b):
    return pl.pallas_call(
        _layer_body,
        grid=(N // BLK,),
        in_specs=[_row_spec(), _col_spec(), _col_spec(),
                  _row_spec(), _col_spec(), _col_spec(),
                  _w_spec(), _b_spec()],
        out_specs=[_row_spec()] * 4,
        out_shape=[jax.ShapeDtypeStruct((N, D), jnp.float32)] * 4,
    )(mf, dinf, doutf, ms, dins, douts, w, b)


def _final_graph(m2, rin, x, h1, w1, b1, wp, bp):
    h2 = jnp.dot(m2 * rin, w1, preferred_element_type=jnp.float32)
    h2 = jnp.maximum(h2 + b1, 0.0)
    h = (jnp.dot(x, wp[0], preferred_element_type=jnp.float32)
         + jnp.dot(h1, wp[1], preferred_element_type=jnp.float32)
         + jnp.dot(h2, wp[2], preferred_element_type=jnp.float32)
         + bp)
    nrm = lax.rsqrt(jnp.sum(h * h, axis=1, keepdims=True))
    return h, h * nrm


def _final_body(m2f_ref, dinf_ref, xf_ref, h1f_ref,
                m2s_ref, dins_ref, xs_ref, h1s_ref,
                w1_ref, b1_ref, wp1a_ref, wp1b_ref, wp1c_ref, bp1_ref,
                wp2a_ref, wp2b_ref, wp2c_ref, bp2_ref,
                hf_ref, hnf_ref, hs_ref, hns_ref):
    w1 = w1_ref[...]
    b1 = b1_ref[...]
    hf_ref[...], hnf_ref[...] = _final_graph(
        m2f_ref[...], lax.rsqrt(jnp.maximum(dinf_ref[...], 1.0)),
        xf_ref[...], h1f_ref[...], w1, b1,
        (wp1a_ref[...], wp1b_ref[...], wp1c_ref[...]), bp1_ref[...])
    hs_ref[...], hns_ref[...] = _final_graph(
        m2s_ref[...], lax.rsqrt(jnp.maximum(dins_ref[...], 1.0)),
        xs_ref[...], h1s_ref[...], w1, b1,
        (wp2a_ref[...], wp2b_ref[...], wp2c_ref[...]), bp2_ref[...])


def _tc_final(m2f, dinf, xf, h1f, m2s, dins, xs, h1s,
              w1, b1, wp1, bp1, wp2, bp2):
    return pl.pallas_call(
        _final_body,
        grid=(N // BLK,),
        in_specs=[_row_spec(), _col_spec(), _row_spec(), _row_spec(),
                  _row_spec(), _col_spec(), _row_spec(), _row_spec(),
                  _w_spec(), _b_spec(),
                  _w_spec(), _w_spec(), _w_spec(), _b_spec(),
                  _w_spec(), _w_spec(), _w_spec(), _b_spec()],
        out_specs=[_row_spec()] * 4,
        out_shape=[jax.ShapeDtypeStruct((N, D), jnp.float32)] * 4,
    )(m2f, dinf, xf, h1f, m2s, dins, xs, h1s,
      w1, b1, wp1[0], wp1[1], wp1[2], bp1, wp2[0], wp2[1], wp2[2], bp2)


PBLK = 4096


def _dot_body(pa_ref, pc_ref, na_ref, nc_ref, po_ref, no_ref):
    po_ref[...] = jnp.sum(pa_ref[...] * pc_ref[...], axis=1, keepdims=True)
    no_ref[...] = jnp.sum(na_ref[...] * nc_ref[...], axis=1, keepdims=True)


def _tc_dot(pa, pc, na, nc):
    rs = pl.BlockSpec((PBLK, D), lambda i: (i, 0))
    cs = pl.BlockSpec((PBLK, 1), lambda i: (i, 0))
    return pl.pallas_call(
        _dot_body,
        grid=(PP // PBLK,),
        in_specs=[rs, rs, rs, rs],
        out_specs=[cs, cs],
        out_shape=[jax.ShapeDtypeStruct((PP, 1), jnp.float32)] * 2,
    )(pa, pc, na, nc)


def kernel(x_first, x_second, edge_index_first, edge_index_second,
           edges_positive_supervision, edges_negative_supervision,
           W0, b0, W1, b1, Wp1, bp1, Wp2, bp2):
    src_f = edge_index_first[0]
    dst_f = edge_index_first[1]
    src_s = edge_index_second[0]
    dst_s = edge_index_second[1]

    deg0, deg1, deg2, deg3 = _sc_deg(src_f, dst_f, src_s, dst_s)
    d_out_f = deg0.reshape(N, 1)
    d_in_f = deg1.reshape(N, 1)
    d_out_s = deg2.reshape(N, 1)
    d_in_s = deg3.reshape(N, 1)

    x0f, x0s = _tc_scale(x_first, d_out_f, x_second, d_out_s)
    m0f, m0s = _sc_segsum(x0f, x0s, src_f, dst_f, src_s, dst_s)
    h1f, x1f, h1s, x1s = _tc_layer(m0f, d_in_f, d_out_f,
                                   m0s, d_in_s, d_out_s, W0, b0)
    m1f, m1s = _sc_segsum(x1f, x1s, src_f, dst_f, src_s, dst_s)
    wp1 = (Wp1[0:D], Wp1[D:2 * D], Wp1[2 * D:3 * D])
    wp2 = (Wp2[0:D], Wp2[D:2 * D], Wp2[2 * D:3 * D])
    hf, hnf, hs, hns = _tc_final(m1f, d_in_f, x_first, h1f,
                                 m1s, d_in_s, x_second, h1s,
                                 W1, b1, wp1, bp1, wp2, bp2)

    pad = ((0, PP - P),)
    p0 = jnp.pad(edges_positive_supervision[0], pad)
    p1 = jnp.pad(edges_positive_supervision[1], pad)
    n0 = jnp.pad(edges_negative_supervision[0], pad)
    n1 = jnp.pad(edges_negative_supervision[1], pad)
    pa, pc, na, nc = _sc_predict_gather(hnf, hns, p0, p1, n0, n1)
    pos, neg = _tc_dot(pa, pc, na, nc)
    return (pos[:P], neg[:P], hf, hs)
